# Initial kernel scaffold; baseline (speedup 1.0000x reference)
import jax
import jax.numpy as jnp
from jax.experimental import pallas as pl


def kernel(feature, idx):
    raise NotImplementedError("write your pallas kernel here")


# speedup vs baseline: 13.1640x; 13.1640x over previous
import functools

import jax
import jax.numpy as jnp
from jax import lax
from jax.experimental import pallas as pl
from jax.experimental.pallas import tpu as pltpu
from jax.experimental.pallas import tpu_sc as plsc

N_OUT = 100000
N_IN = 50000
D = 128

NC = 2
NS = 16
NW = NC * NS
RPW = N_OUT // NW
CH = 128
NZ = -(-RPW // CH)
NBLK = N_IN // 16
MAX_CO = N_IN - CH


def _body(feat_hbm, idx_hbm, out_hbm, idx_v, zbuf, fbuf, idxc, sem_z, sem_s):
    wid = lax.axis_index("s") * NC + lax.axis_index("c")
    start = (wid * RPW) // 8 * 8
    end = ((wid + 1) * RPW) // 8 * 8

    pltpu.sync_copy(idx_hbm, idx_v)

    zeros16 = jnp.zeros((16,), jnp.float32)

    def zrow(r, carry):
        for q in range(D // 16):
            zbuf[r, pl.ds(q * 16, 16)] = zeros16
        return carry

    lax.fori_loop(0, CH, zrow, 0)

    zdescs = []
    for c in range(NZ):
        r0 = jnp.minimum(start + c * CH, end - CH)
        r0 = pl.multiple_of(r0, 8)
        zdescs.append(pltpu.async_copy(zbuf, out_hbm.at[pl.ds(r0, CH)], sem_z))

    def lower_bound(t):
        def step(_, st):
            lo, hi = st
            done = lo >= hi
            m = jnp.minimum((lo + hi) // 2, NBLK - 1)
            v = idx_v[pl.ds(m * 16, 16)]
            go_right = jnp.logical_and(v[0] < t, jnp.logical_not(done))
            go_left = jnp.logical_and(v[0] >= t, jnp.logical_not(done))
            return (jnp.where(go_right, m + 1, lo), jnp.where(go_left, m, hi))

        lo, _ = lax.fori_loop(0, 12, step, (jnp.int32(0), jnp.int32(NBLK)))
        blk = jnp.maximum(lo - 1, 0)
        v = idx_v[pl.ds(blk * 16, 16)]
        cnt = jnp.int32(0)
        for lane in range(16):
            cnt = cnt + jnp.where(v[lane] < t, 1, 0).astype(jnp.int32)
        return blk * 16 + cnt

    s = lower_bound(start)
    e = lower_bound(end)

    for dsc in zdescs:
        dsc.wait()

    s8 = (s // 8) * 8
    nj = (e - s8 + CH - 1) // CH

    def sbody(j, carry):
        co = jnp.minimum(s8 + j * CH, MAX_CO)
        co = pl.multiple_of(co, 8)
        pltpu.sync_copy(idx_hbm.at[pl.ds(co, CH)], idxc)
        pltpu.sync_copy(feat_hbm.at[pl.ds(co, CH)], fbuf)
        pltpu.async_copy(fbuf, out_hbm.at[idxc], sem_s).wait()
        return carry

    lax.fori_loop(0, nj, sbody, 0)


@functools.partial(jax.jit, donate_argnums=())
def _unpool(feat, idx32):
    mesh = plsc.VectorSubcoreMesh(
        core_axis_name="c", subcore_axis_name="s", num_cores=NC, num_subcores=NS
    )
    run = pl.kernel(
        _body,
        out_type=jax.ShapeDtypeStruct((N_OUT, D), jnp.float32),
        mesh=mesh,
        scratch_types=[
            pltpu.VMEM((N_IN,), jnp.int32),
            pltpu.VMEM((CH, D), jnp.float32),
            pltpu.VMEM((CH, D), jnp.float32),
            pltpu.VMEM((CH,), jnp.int32),
            pltpu.SemaphoreType.DMA,
            pltpu.SemaphoreType.DMA,
        ],
    )
    return run(feat, idx32)


def kernel(feature, idx):
    f = feature.reshape(N_IN, D)
    i32 = idx.astype(jnp.int32)
    out = _unpool(f, i32)
    return out.reshape(N_OUT, feature.shape[1], feature.shape[2], feature.shape[3])


# speedup vs baseline: 14.8913x; 1.1312x over previous
import functools

import jax
import jax.numpy as jnp
from jax import lax
from jax.experimental import pallas as pl
from jax.experimental.pallas import tpu as pltpu
from jax.experimental.pallas import tpu_sc as plsc

N_OUT = 100000
N_IN = 50000
D = 128

NC = 2
NS = 16
NW = NC * NS
RPW = N_OUT // NW
CW = 256
CWS = CW + 8
NFULL = 12
NBLK = N_IN // 16


def _body(t_hbm, idx_hbm, out_hbm, idx_v, fsrc, obuf, didx, sem_o):
    wid = lax.axis_index("s") * NC + lax.axis_index("c")
    start = (wid * RPW) // 8 * 8
    end = ((wid + 1) * RPW) // 8 * 8
    tailw = end - start - NFULL * CW

    pltpu.sync_copy(idx_hbm, idx_v.at[pl.ds(0, N_IN)])

    zeros16 = jnp.zeros((16,), jnp.float32)
    lanes = lax.iota(jnp.int32, 16)

    def lower_bound(t):
        def step(_, st):
            lo, hi = st
            done = lo >= hi
            mm = jnp.minimum((lo + hi) // 2, NBLK - 1)
            v = idx_v[pl.ds(mm * 16, 16)]
            go_right = jnp.logical_and(v[0] < t, jnp.logical_not(done))
            go_left = jnp.logical_and(v[0] >= t, jnp.logical_not(done))
            return (jnp.where(go_right, mm + 1, lo), jnp.where(go_left, mm, hi))

        lo, _ = lax.fori_loop(0, 12, step, (jnp.int32(0), jnp.int32(NBLK)))
        blk = jnp.maximum(lo - 1, 0)
        v = idx_v[pl.ds(blk * 16, 16)]
        cnt = jnp.int32(0)
        for lane in range(16):
            cnt = cnt + jnp.where(v[lane] < t, 1, 0).astype(jnp.int32)
        return blk * 16 + cnt

    out_desc = None
    for j in range(NFULL + 1):
        full = j < NFULL
        zq = CW // 16 if full else 4
        cb = start + j * CW
        cb = pl.multiple_of(cb, 8)
        cwj = CW if full else tailw

        s_c = lower_bound(cb)
        e_c = lower_bound(cb + cwj)
        m = e_c - s_c

        sf = jnp.clip(s_c - s_c % 8, 0, N_IN - CWS)
        sf = pl.multiple_of(sf, 8)
        soff = s_c - sf
        pltpu.sync_copy(t_hbm.at[:, pl.ds(sf, CWS)], fsrc)

        for u in range(CW // 16):
            didx[pl.ds(u * 16, 16)] = idx_v[pl.ds(s_c + u * 16, 16)] - cb

        if out_desc is not None:
            out_desc.wait()
            out_desc = None

        def zrow(p, carry):
            for q in range(zq):
                obuf[p, pl.ds(q * 16, 16)] = zeros16
            return carry

        lax.fori_loop(0, D, zrow, 0, unroll=2)

        um = (m + 15) // 16

        def ubody(u, carry):
            di = didx[pl.ds(u * 16, 16)]
            msk = (lanes + u * 16) < m
            so = soff + u * 16

            def pbody(p, c2):
                v = fsrc[p, pl.ds(so, 16)]
                plsc.store_scatter(obuf, [jnp.full((16,), p, jnp.int32), di], v, mask=msk)
                return c2

            lax.fori_loop(0, D, pbody, 0, unroll=4)
            return carry

        lax.fori_loop(0, um, ubody, 0)

        if full:
            out_desc = pltpu.async_copy(obuf, out_hbm.at[:, pl.ds(cb, CW)], sem_o)
        else:
            @pl.when(tailw == 48)
            def _():
                pltpu.sync_copy(obuf.at[:, pl.ds(0, 48)], out_hbm.at[:, pl.ds(cb, 48)])

            @pl.when(tailw == 56)
            def _():
                pltpu.sync_copy(obuf.at[:, pl.ds(0, 56)], out_hbm.at[:, pl.ds(cb, 56)])


@jax.jit
def _unpool(t2, idx32):
    mesh = plsc.VectorSubcoreMesh(
        core_axis_name="c", subcore_axis_name="s", num_cores=NC, num_subcores=NS
    )
    run = pl.kernel(
        _body,
        out_type=jax.ShapeDtypeStruct((D, N_OUT), jnp.float32),
        mesh=mesh,
        scratch_types=[
            pltpu.VMEM((N_IN + CW,), jnp.int32),
            pltpu.VMEM((D, CWS), jnp.float32),
            pltpu.VMEM((D, CW), jnp.float32),
            pltpu.VMEM((CW,), jnp.int32),
            pltpu.SemaphoreType.DMA,
        ],
        compiler_params=pltpu.CompilerParams(use_tc_tiling_on_sc=False, needs_layout_passes=False),
    )
    return run(t2, idx32)


def kernel(feature, idx):
    t2 = jnp.squeeze(feature, 3).transpose(1, 2, 0).reshape(D, N_IN)
    out2 = _unpool(t2, idx.astype(jnp.int32))
    return out2.reshape(64, 2, N_OUT).transpose(2, 0, 1)[:, :, :, None]


# speedup vs baseline: 15.2492x; 1.0240x over previous
import functools

import jax
import jax.numpy as jnp
from jax import lax
from jax.experimental import pallas as pl
from jax.experimental.pallas import tpu as pltpu
from jax.experimental.pallas import tpu_sc as plsc

N_OUT = 100000
N_IN = 50000
D = 128

NC = 2
NS = 16
NW = NC * NS
RPW = N_OUT // NW
CW = 128
CWS = CW + 8
NFULL = 24
NBLK = N_IN // 16


def _body(t_hbm, idx_hbm, out_hbm, idx_v, fsrc, obuf, zb, didx, sem_o, sem_f):
    sid = lax.axis_index("s")
    wid = lax.axis_index("s") * NC + lax.axis_index("c")
    start = (wid * RPW) // 8 * 8
    end = ((wid + 1) * RPW) // 8 * 8
    tailw = end - start - NFULL * CW

    pltpu.sync_copy(idx_hbm, idx_v.at[pl.ds(0, N_IN)])

    zeros16 = jnp.zeros((16,), jnp.float32)
    lanes = lax.iota(jnp.int32, 16)

    def zbrow(p, carry):
        for q in range(CW // 16):
            obuf[p, pl.ds(q * 16, 16)] = zeros16
        return carry

    lax.fori_loop(0, D, zbrow, 0, unroll=2)
    pltpu.sync_copy(obuf, zb.at[sid])

    def lower_bound(t):
        def step(_, st):
            lo, hi = st
            done = lo >= hi
            mm = jnp.minimum((lo + hi) // 2, NBLK - 1)
            v = idx_v[pl.ds(mm * 16, 16)]
            go_right = jnp.logical_and(v[0] < t, jnp.logical_not(done))
            go_left = jnp.logical_and(v[0] >= t, jnp.logical_not(done))
            return (jnp.where(go_right, mm + 1, lo), jnp.where(go_left, mm, hi))

        lo, _ = lax.fori_loop(0, 12, step, (jnp.int32(0), jnp.int32(NBLK)))
        blk = jnp.maximum(lo - 1, 0)
        v = idx_v[pl.ds(blk * 16, 16)]
        cnt = jnp.int32(0)
        for lane in range(16):
            cnt = cnt + jnp.where(v[lane] < t, 1, 0).astype(jnp.int32)
        return blk * 16 + cnt

    def bounds(j):
        cwj = CW if j < NFULL else tailw
        cb = pl.multiple_of(start + j * CW, 8)
        s_c = lower_bound(cb)
        e_c = lower_bound(cb + cwj)
        sf = jnp.clip(s_c - s_c % 8, 0, N_IN - CWS)
        sf = pl.multiple_of(sf, 8)
        return cb, s_c, e_c - s_c, sf

    fdesc = [None, None]
    binfo = [None, None]

    def prefetch(j):
        p = j % 2
        binfo[p] = bounds(j)
        _, _, _, sf = binfo[p]
        fdesc[p] = pltpu.async_copy(t_hbm.at[:, pl.ds(sf, CWS)], fsrc.at[p], sem_f)

    prefetch(0)
    out_desc = None
    for j in range(NFULL + 1):
        full = j < NFULL
        pb = j % 2
        cb, s_c, m, sf = binfo[pb]
        soff = s_c - sf
        if j < NFULL:
            prefetch(j + 1)

        for u in range(CW // 16):
            didx[pl.ds(u * 16, 16)] = idx_v[pl.ds(s_c + u * 16, 16)] - cb

        if out_desc is not None:
            out_desc.wait()
            out_desc = None
        pltpu.sync_copy(zb.at[sid], obuf)

        fdesc[pb].wait()
        um = (m + 15) // 16

        def ubody(u, carry):
            di = didx[pl.ds(u * 16, 16)]
            msk = (lanes + u * 16) < m
            so = soff + u * 16

            def pbody(p, c2):
                v = fsrc[pb, p, pl.ds(so, 16)]
                plsc.store_scatter(obuf.at[p], [di], v, mask=msk)
                return c2

            lax.fori_loop(0, D, pbody, 0, unroll=4)
            return carry

        lax.fori_loop(0, um, ubody, 0)

        if full:
            out_desc = pltpu.async_copy(obuf, out_hbm.at[:, pl.ds(cb, CW)], sem_o)
        else:
            @pl.when(tailw == 48)
            def _():
                pltpu.sync_copy(obuf.at[:, pl.ds(0, 48)], out_hbm.at[:, pl.ds(cb, 48)])

            @pl.when(tailw == 56)
            def _():
                pltpu.sync_copy(obuf.at[:, pl.ds(0, 56)], out_hbm.at[:, pl.ds(cb, 56)])


@jax.jit
def _unpool(t2, idx32):
    mesh = plsc.VectorSubcoreMesh(
        core_axis_name="c", subcore_axis_name="s", num_cores=NC, num_subcores=NS
    )
    run = pl.kernel(
        _body,
        out_type=jax.ShapeDtypeStruct((D, N_OUT), jnp.float32),
        mesh=mesh,
        scratch_types=[
            pltpu.VMEM((N_IN + CW,), jnp.int32),
            pltpu.VMEM((2, D, CWS), jnp.float32),
            pltpu.VMEM((D, CW), jnp.float32),
            pltpu.VMEM_SHARED((NS, D, CW), jnp.float32),
            pltpu.VMEM((CW,), jnp.int32),
            pltpu.SemaphoreType.DMA,
            pltpu.SemaphoreType.DMA,
        ],
        compiler_params=pltpu.CompilerParams(use_tc_tiling_on_sc=False, needs_layout_passes=False),
    )
    return run(t2, idx32)


def kernel(feature, idx):
    t2 = jnp.squeeze(feature, 3).transpose(1, 2, 0).reshape(D, N_IN)
    out2 = _unpool(t2, idx.astype(jnp.int32))
    return out2.reshape(64, 2, N_OUT).transpose(2, 0, 1)[:, :, :, None]


# speedup vs baseline: 16.3051x; 1.0692x over previous
import functools

import jax
import jax.numpy as jnp
from jax import lax
from jax.experimental import pallas as pl
from jax.experimental.pallas import tpu as pltpu
from jax.experimental.pallas import tpu_sc as plsc

N_OUT = 100000
N_IN = 50000
D = 128

NC = 2
NS = 16
NW = NC * NS
RPW = N_OUT // NW
CW = 128
CWS = CW + 8
NFULL = 24
NBLK = N_IN // 16


def _body(t_hbm, idx_hbm, out_hbm, idx_v, fsrc, obuf, zb, didx, sem_o, sem_f, sem_z):
    wid = lax.axis_index("s") * NC + lax.axis_index("c")
    start = (wid * RPW) // 8 * 8
    end = ((wid + 1) * RPW) // 8 * 8
    tailw = end - start - NFULL * CW

    pltpu.sync_copy(idx_hbm, idx_v.at[pl.ds(0, N_IN)])

    zeros16 = jnp.zeros((16,), jnp.float32)
    lanes = lax.iota(jnp.int32, 16)

    def zbrow(p, carry):
        for q in range(CW // 16):
            obuf[0, p, pl.ds(q * 16, 16)] = zeros16
        return carry

    lax.fori_loop(0, D, zbrow, 0, unroll=2)
    pltpu.sync_copy(obuf.at[0], zb)
    zdesc = [None, pltpu.async_copy(zb, obuf.at[1], sem_z)]

    def lower_bound(t):
        def step(_, st):
            lo, hi = st
            done = lo >= hi
            mm = jnp.minimum((lo + hi) // 2, NBLK - 1)
            v = idx_v[pl.ds(mm * 16, 16)]
            go_right = jnp.logical_and(v[0] < t, jnp.logical_not(done))
            go_left = jnp.logical_and(v[0] >= t, jnp.logical_not(done))
            return (jnp.where(go_right, mm + 1, lo), jnp.where(go_left, mm, hi))

        lo, _ = lax.fori_loop(0, 12, step, (jnp.int32(0), jnp.int32(NBLK)))
        blk = jnp.maximum(lo - 1, 0)
        v = idx_v[pl.ds(blk * 16, 16)]
        cnt = jnp.int32(0)
        for lane in range(16):
            cnt = cnt + jnp.where(v[lane] < t, 1, 0).astype(jnp.int32)
        return blk * 16 + cnt

    def bounds(j):
        cwj = CW if j < NFULL else tailw
        cb = pl.multiple_of(start + j * CW, 8)
        s_c = lower_bound(cb)
        e_c = lower_bound(cb + cwj)
        sf = jnp.clip(s_c - s_c % 8, 0, N_IN - CWS)
        sf = pl.multiple_of(sf, 8)
        return cb, s_c, e_c - s_c, sf

    fdesc = [None, None]
    binfo = [None, None]

    def prefetch(j):
        p = j % 2
        binfo[p] = bounds(j)
        _, _, _, sf = binfo[p]
        fdesc[p] = pltpu.async_copy(t_hbm.at[:, pl.ds(sf, CWS)], fsrc.at[p], sem_f)

    prefetch(0)
    odesc = [None, None]
    for j in range(NFULL + 1):
        full = j < NFULL
        pb = j % 2
        cb, s_c, m, sf = binfo[pb]
        soff = s_c - sf
        if j < NFULL:
            prefetch(j + 1)

        for u in range(CW // 16):
            didx[pl.ds(u * 16, 16)] = idx_v[pl.ds(s_c + u * 16, 16)] - cb

        if zdesc[pb] is not None:
            zdesc[pb].wait()
            zdesc[pb] = None
        fdesc[pb].wait()
        um = (m + 15) // 16

        def ubody(u, carry):
            di = didx[pl.ds(u * 16, 16)]
            msk = (lanes + u * 16) < m
            so = soff + u * 16

            def pbody(p, c2):
                v = fsrc[pb, p, pl.ds(so, 16)]
                plsc.store_scatter(obuf.at[pb, p], [di], v, mask=msk)
                return c2

            lax.fori_loop(0, D, pbody, 0, unroll=8)
            return carry

        lax.fori_loop(0, um, ubody, 0)

        if full:
            odesc[pb] = pltpu.async_copy(obuf.at[pb], out_hbm.at[:, pl.ds(cb, CW)], sem_o)
        else:
            @pl.when(tailw == 48)
            def _():
                pltpu.sync_copy(obuf.at[pb, :, pl.ds(0, 48)], out_hbm.at[:, pl.ds(cb, 48)])

            @pl.when(tailw == 56)
            def _():
                pltpu.sync_copy(obuf.at[pb, :, pl.ds(0, 56)], out_hbm.at[:, pl.ds(cb, 56)])

        if 1 <= j <= NFULL - 1:
            odesc[1 - pb].wait()
            zdesc[1 - pb] = pltpu.async_copy(zb, obuf.at[1 - pb], sem_z)
    odesc[(NFULL - 1) % 2].wait()


@jax.jit
def _unpool(t2, idx32):
    mesh = plsc.VectorSubcoreMesh(
        core_axis_name="c", subcore_axis_name="s", num_cores=NC, num_subcores=NS
    )
    run = pl.kernel(
        _body,
        out_type=jax.ShapeDtypeStruct((D, N_OUT), jnp.float32),
        mesh=mesh,
        scratch_types=[
            pltpu.VMEM((N_IN + CW,), jnp.int32),
            pltpu.VMEM((2, D, CWS), jnp.float32),
            pltpu.VMEM((2, D, CW), jnp.float32),
            pltpu.VMEM_SHARED((D, CW), jnp.float32),
            pltpu.VMEM((CW,), jnp.int32),
            pltpu.SemaphoreType.DMA,
            pltpu.SemaphoreType.DMA,
            pltpu.SemaphoreType.DMA,
        ],
        compiler_params=pltpu.CompilerParams(use_tc_tiling_on_sc=False, needs_layout_passes=False),
    )
    return run(t2, idx32)


def kernel(feature, idx):
    t2 = jnp.squeeze(feature, 3).transpose(1, 2, 0).reshape(D, N_IN)
    out2 = _unpool(t2, idx.astype(jnp.int32))
    return out2.reshape(64, 2, N_OUT).transpose(2, 0, 1)[:, :, :, None]


# speedup vs baseline: 16.4215x; 1.0071x over previous
import functools

import jax
import jax.numpy as jnp
from jax import lax
from jax.experimental import pallas as pl
from jax.experimental.pallas import tpu as pltpu
from jax.experimental.pallas import tpu_sc as plsc

N_OUT = 100000
N_IN = 50000
D = 128

NC = 2
NS = 16
NW = NC * NS
RPW = N_OUT // NW
CW = 128
CWS = CW + 8
NFULL = 24
NBLK = N_IN // 16


def _body(t_hbm, idx_hbm, out_hbm, idx_v, fsrc, obuf, zb, didx, sem_o, sem_f, sem_z):
    wid = lax.axis_index("s") * NC + lax.axis_index("c")
    start = (wid * RPW) // 8 * 8
    end = ((wid + 1) * RPW) // 8 * 8
    tailw = end - start - NFULL * CW

    pltpu.sync_copy(idx_hbm, idx_v.at[pl.ds(0, N_IN)])

    zeros16 = jnp.zeros((16,), jnp.float32)
    lanes = lax.iota(jnp.int32, 16)

    def zbrow(p, carry):
        for q in range(CW // 16):
            obuf[0, p, pl.ds(q * 16, 16)] = zeros16
        return carry

    lax.fori_loop(0, D, zbrow, 0, unroll=2)
    pltpu.sync_copy(obuf.at[0], zb)
    zdesc = [None, pltpu.async_copy(zb, obuf.at[1], sem_z)]

    def lower_bound(t):
        def step(_, st):
            lo, hi = st
            done = lo >= hi
            mm = jnp.minimum((lo + hi) // 2, NBLK - 1)
            v = idx_v[pl.ds(mm * 16, 16)]
            go_right = jnp.logical_and(v[0] < t, jnp.logical_not(done))
            go_left = jnp.logical_and(v[0] >= t, jnp.logical_not(done))
            return (jnp.where(go_right, mm + 1, lo), jnp.where(go_left, mm, hi))

        lo, _ = lax.fori_loop(0, 12, step, (jnp.int32(0), jnp.int32(NBLK)))
        blk = jnp.maximum(lo - 1, 0)
        v = idx_v[pl.ds(blk * 16, 16)]
        cnt = jnp.int32(0)
        for lane in range(16):
            cnt = cnt + jnp.where(v[lane] < t, 1, 0).astype(jnp.int32)
        return blk * 16 + cnt

    bound_carry = [lower_bound(start)]

    def bounds(j):
        cwj = CW if j < NFULL else tailw
        cb = pl.multiple_of(start + j * CW, 8)
        s_c = bound_carry[0]
        e_c = lower_bound(cb + cwj)
        bound_carry[0] = e_c
        sf = jnp.clip(s_c - s_c % 8, 0, N_IN - CWS)
        sf = pl.multiple_of(sf, 8)
        return cb, s_c, e_c - s_c, sf

    fdesc = [None, None]
    binfo = [None, None]

    def prefetch(j):
        p = j % 2
        binfo[p] = bounds(j)
        _, _, _, sf = binfo[p]
        fdesc[p] = pltpu.async_copy(t_hbm.at[:, pl.ds(sf, CWS)], fsrc.at[p], sem_f)

    prefetch(0)
    odesc = [None, None]
    for j in range(NFULL + 1):
        full = j < NFULL
        pb = j % 2
        cb, s_c, m, sf = binfo[pb]
        soff = s_c - sf
        if j < NFULL:
            prefetch(j + 1)

        for u in range(CW // 16):
            didx[pl.ds(u * 16, 16)] = idx_v[pl.ds(s_c + u * 16, 16)] - cb

        if zdesc[pb] is not None:
            zdesc[pb].wait()
            zdesc[pb] = None
        fdesc[pb].wait()
        um = (m + 15) // 16

        def ubody(u, carry):
            di = didx[pl.ds(u * 16, 16)]
            msk = (lanes + u * 16) < m
            so = soff + u * 16

            def pbody(p, c2):
                v = fsrc[pb, p, pl.ds(so, 16)]
                plsc.store_scatter(obuf.at[pb, p], [di], v, mask=msk)
                return c2

            lax.fori_loop(0, D, pbody, 0, unroll=8)
            return carry

        lax.fori_loop(0, um, ubody, 0)

        if full:
            odesc[pb] = pltpu.async_copy(obuf.at[pb], out_hbm.at[:, pl.ds(cb, CW)], sem_o)
        else:
            @pl.when(tailw == 48)
            def _():
                pltpu.sync_copy(obuf.at[pb, :, pl.ds(0, 48)], out_hbm.at[:, pl.ds(cb, 48)])

            @pl.when(tailw == 56)
            def _():
                pltpu.sync_copy(obuf.at[pb, :, pl.ds(0, 56)], out_hbm.at[:, pl.ds(cb, 56)])

        if 1 <= j <= NFULL - 1:
            odesc[1 - pb].wait()
            zdesc[1 - pb] = pltpu.async_copy(zb, obuf.at[1 - pb], sem_z)
    odesc[(NFULL - 1) % 2].wait()


@jax.jit
def _unpool(t2, idx32):
    mesh = plsc.VectorSubcoreMesh(
        core_axis_name="c", subcore_axis_name="s", num_cores=NC, num_subcores=NS
    )
    run = pl.kernel(
        _body,
        out_type=jax.ShapeDtypeStruct((D, N_OUT), jnp.float32),
        mesh=mesh,
        scratch_types=[
            pltpu.VMEM((N_IN + CW,), jnp.int32),
            pltpu.VMEM((2, D, CWS), jnp.float32),
            pltpu.VMEM((2, D, CW), jnp.float32),
            pltpu.VMEM_SHARED((D, CW), jnp.float32),
            pltpu.VMEM((CW,), jnp.int32),
            pltpu.SemaphoreType.DMA,
            pltpu.SemaphoreType.DMA,
            pltpu.SemaphoreType.DMA,
        ],
        compiler_params=pltpu.CompilerParams(use_tc_tiling_on_sc=False, needs_layout_passes=False),
    )
    return run(t2, idx32)


def kernel(feature, idx):
    t2 = jnp.squeeze(feature, 3).transpose(1, 2, 0).reshape(D, N_IN)
    out2 = _unpool(t2, idx.astype(jnp.int32))
    return out2.reshape(64, 2, N_OUT).transpose(2, 0, 1)[:, :, :, None]
